# baseline (speedup 1.0000x reference)
import functools

import jax
import jax.numpy as jnp
from jax import lax
from jax.experimental import pallas as pl
from jax.experimental.pallas import tpu as pltpu
from jax.experimental.pallas import tpu_sc as plsc

NUM_EMB = 100001
WIDTH = 64
BATCH = 16384

_info = plsc.get_sparse_core_info()
_NC, _NS = _info.num_cores, _info.num_subcores
_NW = _NC * _NS
_BPW = BATCH // _NW
_C = 3456
_NCH = -(-NUM_EMB // _C)
_RPW = _BPW + 1


@functools.partial(
    pl.kernel,
    mesh=plsc.VectorSubcoreMesh(core_axis_name="c", subcore_axis_name="s"),
    out_type=jax.ShapeDtypeStruct((BATCH, WIDTH), jnp.float32),
    scratch_types=[
        pltpu.VMEM_SHARED((_C, WIDTH), jnp.float32),
        pltpu.VMEM_SHARED((_C, WIDTH), jnp.float32),
        pltpu.VMEM_SHARED((_NS * _RPW, WIDTH), jnp.float32),
        pltpu.VMEM((_BPW,), jnp.int32),
        pltpu.VMEM((_BPW + 16,), jnp.int32),
        pltpu.VMEM((_BPW + 16,), jnp.int32),
        pltpu.VMEM((16, WIDTH), jnp.float32),
        pltpu.SemaphoreType.DMA,
    ],
    compiler_params=pltpu.CompilerParams(needs_layout_passes=False),
)
def _embed(idx_hbm, table_hbm, out_hbm, sp_a, sp_b, sp_rows, idx_v, lloc,
           lpos, stg, sem):
    c = lax.axis_index("c")
    s = lax.axis_index("s")
    wid = c * _NS + s
    base = wid * _BPW
    pltpu.sync_copy(idx_hbm.at[pl.ds(base, _BPW)], idx_v)

    lanes = lax.iota(jnp.int32, 16)
    bufs = [sp_a, sp_b]

    def stage(k):
        lo = k * _C
        n = min(_C, NUM_EMB - lo)
        per = 8 * (-(-n // (8 * _NS)))
        last = 8 * (-(-(n - per) // 8))
        a = pl.multiple_of(jnp.minimum(s * per, last), 8)
        return pltpu.async_copy(
            table_hbm.at[pl.ds(lo + a, per)],
            bufs[k % 2].at[pl.ds(a, per)],
            sem,
        )

    pending = [None, None]
    pending[0] = stage(0)

    for k in range(_NCH):
        lo = k * _C
        n = min(_C, NUM_EMB - lo)
        buf = bufs[k % 2]

        pending[k % 2].wait()
        plsc.subcore_barrier()

        if k + 1 < _NCH:
            pending[(k + 1) % 2] = stage(k + 1)

        def scan_body(v, cur):
            idx16 = idx_v[pl.ds(v * 16, 16)]
            m = (idx16 >= lo) & (idx16 < lo + n)
            key = jnp.where(m, idx16 - lo, jnp.int32(1 << 30))
            pos = s * _RPW + v * 16 + lanes
            skey, spos = lax.sort((key, pos), num_keys=1)
            lloc[pl.ds(cur, 16)] = skey
            lpos[pl.ds(cur, 16)] = spos
            return cur + jnp.sum(m.astype(jnp.int32))

        cnt = lax.fori_loop(0, _BPW // 16, scan_body, 0)

        def group_body(g, _):
            valid = g * 16 + lanes < cnt
            l16 = jnp.where(valid, lloc[pl.ds(g * 16, 16)], 0)
            p16 = jnp.where(valid, lpos[pl.ds(g * 16, 16)], s * _RPW + _BPW)
            pltpu.sync_copy(buf.at[l16], stg)
            pltpu.sync_copy(stg, sp_rows.at[p16])
            return 0

        lax.fori_loop(0, (cnt + 15) // 16, group_body, 0)

        plsc.subcore_barrier()

    pltpu.sync_copy(
        sp_rows.at[pl.ds(s * _RPW, _BPW)], out_hbm.at[pl.ds(base, _BPW)]
    )


def kernel(x, table):
    return _embed(x.astype(jnp.int32), table)


# speedup vs baseline: 1.2244x; 1.2244x over previous
import functools

import jax
import jax.numpy as jnp
from jax import lax
from jax.experimental import pallas as pl
from jax.experimental.pallas import tpu as pltpu
from jax.experimental.pallas import tpu_sc as plsc

NUM_EMB = 100001
WIDTH = 64
BATCH = 16384
PAIRS = (NUM_EMB + 1) // 2

_info = plsc.get_sparse_core_info()
_NC, _NS = _info.num_cores, _info.num_subcores
_NW = _NC * _NS
_BPW = BATCH // _NW
_G = 128
_NG = _BPW // _G


@functools.partial(
    pl.kernel,
    mesh=plsc.VectorSubcoreMesh(core_axis_name="c", subcore_axis_name="s"),
    out_type=jax.ShapeDtypeStruct((BATCH, WIDTH), jnp.float32),
    scratch_types=[
        pltpu.VMEM((_BPW,), jnp.int32),
        pltpu.VMEM((_NG, _G), jnp.int32),
        pltpu.VMEM((_G, 2 * WIDTH), jnp.float32),
        pltpu.VMEM((_G, 2 * WIDTH), jnp.float32),
        pltpu.VMEM((_G, WIDTH), jnp.float32),
        pltpu.VMEM((_G, WIDTH), jnp.float32),
        pltpu.SemaphoreType.DMA,
        pltpu.SemaphoreType.DMA,
    ],
    compiler_params=pltpu.CompilerParams(needs_layout_passes=False),
)
def _embed(idx_hbm, tp_hbm, out_hbm, idx_v, pidx, pra, prb, sela, selb,
           semg, semw):
    c = lax.axis_index("c")
    s = lax.axis_index("s")
    wid = c * _NS + s
    base = wid * _BPW
    pltpu.sync_copy(idx_hbm.at[pl.ds(base, _BPW)], idx_v)

    lanes = lax.iota(jnp.int32, 16)
    prs = [pra, prb]
    sels = [sela, selb]

    for v in range(_BPW // 16):
        idx16 = idx_v[pl.ds(v * 16, 16)]
        pidx[v * 16 // _G, pl.ds(v * 16 % _G, 16)] = idx16 >> 1

    def gather(j):
        return pltpu.async_copy(tp_hbm.at[pidx.at[j]], prs[j % 2], semg)

    def write(j):
        return pltpu.async_copy(
            sels[j % 2], out_hbm.at[pl.ds(base + j * _G, _G)], semw
        )

    def select(j):
        pr, sel = prs[j % 2], sels[j % 2]

        def row_body(r, _):
            grp = idx_v[pl.ds(j * _G + 16 * (r // 16), 16)]
            lane = jnp.broadcast_to(r % 16, (16,)).astype(jnp.int32)
            xval = lax.gather(
                grp, lane[:, None],
                lax.GatherDimensionNumbers(
                    offset_dims=(), collapsed_slice_dims=(0,),
                    start_index_map=(0,)),
                (1,), mode=lax.GatherScatterMode.PROMISE_IN_BOUNDS)
            off = (xval & 1) * WIDTH
            row = jnp.broadcast_to(r, (16,)).astype(jnp.int32)
            for w in range(WIDTH // 16):
                vals = plsc.load_gather(pr, [row, off + w * 16 + lanes])
                plsc.store_scatter(sel, [row, w * 16 + lanes], vals)
            return 0

        lax.fori_loop(0, _G, row_body, 0)

    gathers = [gather(0)]
    writes = [None] * _NG
    for j in range(_NG):
        gathers[j].wait()
        if j + 1 < _NG:
            gathers.append(gather(j + 1))
        if j >= 2:
            writes[j - 2].wait()
        select(j)
        writes[j] = write(j)
    writes[_NG - 2].wait()
    writes[_NG - 1].wait()


def kernel(x, table):
    tp = jnp.concatenate(
        [table.reshape(-1), jnp.zeros((WIDTH,), jnp.float32)]
    ).reshape(PAIRS, 2 * WIDTH)
    return _embed(x.astype(jnp.int32), tp)


# speedup vs baseline: 1.3973x; 1.1412x over previous
import functools

import jax
import jax.numpy as jnp
from jax import lax
from jax.experimental import pallas as pl
from jax.experimental.pallas import tpu as pltpu
from jax.experimental.pallas import tpu_sc as plsc

NUM_EMB = 100001
WIDTH = 64
BATCH = 16384
PAIRS = (NUM_EMB + 1) // 2

_info = plsc.get_sparse_core_info()
_NC, _NS = _info.num_cores, _info.num_subcores
_NW = _NC * _NS
_BPW = BATCH // _NW
_G = 128
_NG = _BPW // _G


@functools.partial(
    pl.kernel,
    mesh=plsc.VectorSubcoreMesh(core_axis_name="c", subcore_axis_name="s"),
    out_type=jax.ShapeDtypeStruct((BATCH, WIDTH), jnp.float32),
    scratch_types=[
        pltpu.VMEM((_BPW,), jnp.int32),
        pltpu.VMEM((_NG, _G), jnp.int32),
        pltpu.VMEM((1, WIDTH), jnp.float32),
        pltpu.VMEM((_G, 2 * WIDTH), jnp.float32),
        pltpu.VMEM((_G, 2 * WIDTH), jnp.float32),
        pltpu.VMEM((_G, WIDTH), jnp.float32),
        pltpu.VMEM((_G, WIDTH), jnp.float32),
        pltpu.SemaphoreType.DMA,
        pltpu.SemaphoreType.DMA,
    ],
    compiler_params=pltpu.CompilerParams(needs_layout_passes=False),
)
def _embed(idx_hbm, tp_hbm, spec_hbm, out_hbm, idx_v, pidx, spec_v, pra, prb,
           sela, selb, semg, semw):
    c = lax.axis_index("c")
    s = lax.axis_index("s")
    wid = c * _NS + s
    base = wid * _BPW
    pltpu.sync_copy(idx_hbm.at[pl.ds(base, _BPW)], idx_v)
    pltpu.sync_copy(spec_hbm, spec_v)

    lanes = lax.iota(jnp.int32, 16)
    prs = [pra, prb]
    sels = [sela, selb]

    for v in range(_BPW // 16):
        idx16 = idx_v[pl.ds(v * 16, 16)]
        pidx[v * 16 // _G, pl.ds(v * 16 % _G, 16)] = jnp.minimum(
            idx16 >> 1, PAIRS - 2
        )

    def gather(j):
        return pltpu.async_copy(tp_hbm.at[pidx.at[j]], prs[j % 2], semg)

    def write(j):
        return pltpu.async_copy(
            sels[j % 2], out_hbm.at[pl.ds(base + j * _G, _G)], semw
        )

    def select(j):
        pr, sel = prs[j % 2], sels[j % 2]

        def row_body(r, _):
            grp = idx_v[pl.ds(j * _G + 16 * (r // 16), 16)]
            lane = jnp.broadcast_to(r % 16, (16,)).astype(jnp.int32)
            xval = lax.gather(
                grp, lane[:, None],
                lax.GatherDimensionNumbers(
                    offset_dims=(), collapsed_slice_dims=(0,),
                    start_index_map=(0,)),
                (1,), mode=lax.GatherScatterMode.PROMISE_IN_BOUNDS)
            off = (xval & 1) * WIDTH
            row = jnp.broadcast_to(r, (16,)).astype(jnp.int32)
            for w in range(WIDTH // 16):
                vals = plsc.load_gather(pr, [row, off + w * 16 + lanes])
                vals = jnp.where(
                    xval >= NUM_EMB - 1, spec_v[0, pl.ds(w * 16, 16)], vals
                )
                plsc.store_scatter(sel, [row, w * 16 + lanes], vals)
            return 0

        lax.fori_loop(0, _G, row_body, 0)

    gathers = [gather(0)]
    writes = [None] * _NG
    for j in range(_NG):
        gathers[j].wait()
        if j + 1 < _NG:
            gathers.append(gather(j + 1))
        if j >= 2:
            writes[j - 2].wait()
        select(j)
        writes[j] = write(j)
    writes[_NG - 2].wait()
    writes[_NG - 1].wait()


def kernel(x, table):
    tp = lax.slice(table, (0, 0), (NUM_EMB - 1, 0 + WIDTH)).reshape(
        PAIRS - 1, 2 * WIDTH
    )
    spec = lax.slice(table, (NUM_EMB - 1, 0), (NUM_EMB, WIDTH))
    return _embed(x.astype(jnp.int32), tp, spec)


# speedup vs baseline: 1.5091x; 1.0801x over previous
import functools

import jax
import jax.numpy as jnp
from jax import lax
from jax.experimental import pallas as pl
from jax.experimental.pallas import tpu as pltpu
from jax.experimental.pallas import tpu_sc as plsc

NUM_EMB = 100001
WIDTH = 64
BATCH = 16384

_info = plsc.get_sparse_core_info()
_NC, _NS = _info.num_cores, _info.num_subcores
_NW = _NC * _NS
_BPW = BATCH // _NW
_CHUNK = 128
_NCHUNK = _BPW // _CHUNK


@functools.partial(
    pl.kernel,
    mesh=plsc.VectorSubcoreMesh(core_axis_name="c", subcore_axis_name="s"),
    out_type=jax.ShapeDtypeStruct((BATCH, WIDTH), jnp.float32),
    scratch_types=[
        pltpu.VMEM((_BPW,), jnp.int32),
        pltpu.VMEM((_BPW, WIDTH), jnp.float32),
        pltpu.SemaphoreType.DMA,
    ],
    compiler_params=pltpu.CompilerParams(use_tc_tiling_on_sc=False),
)
def _embed(idx_hbm, table_hbm, out_hbm, idx_v, rows_v, sem):
    wid = lax.axis_index("s") * _NC + lax.axis_index("c")
    base = wid * _BPW
    pltpu.sync_copy(idx_hbm.at[pl.ds(base, _BPW)], idx_v)
    copies = []
    for j in range(_NCHUNK):
        copies.append(
            pltpu.async_copy(
                table_hbm.at[idx_v.at[pl.ds(j * _CHUNK, _CHUNK)]],
                rows_v.at[pl.ds(j * _CHUNK, _CHUNK)],
                sem,
            )
        )
    for c in copies:
        c.wait()
    pltpu.sync_copy(rows_v, out_hbm.at[pl.ds(base, _BPW)])


def kernel(x, table):
    return _embed(x.astype(jnp.int32), table)
